# baseline (speedup 1.0000x reference)
import functools

import jax
import jax.numpy as jnp
from jax import lax
from jax.experimental import pallas as pl
from jax.experimental.pallas import tpu as pltpu
from jax.experimental.pallas import tpu_sc as plsc

N_NODES = 10000
D = 128
E = 320000
NC = 2
NS = 16
L = 16
NW = NC * NS
E_PER_W = E // NW
CH = 80
DW = D // 2
N_CH = E_PER_W // CH
G_PER_CH = CH // L
NBUF = 4

_mesh = plsc.VectorSubcoreMesh(core_axis_name="c", subcore_axis_name="s")


@functools.partial(
    pl.kernel,
    out_type=jax.ShapeDtypeStruct((E,), jnp.float32),
    mesh=_mesh,
    scratch_types=[
        pltpu.VMEM((E_PER_W,), jnp.int32),
        pltpu.VMEM((E_PER_W,), jnp.int32),
        [pltpu.VMEM((CH, DW), jnp.int32) for _ in range(NBUF)],
        [pltpu.VMEM((CH, DW), jnp.int32) for _ in range(NBUF)],
        pltpu.VMEM((E_PER_W,), jnp.float32),
        [pltpu.SemaphoreType.DMA for _ in range(NBUF)],
        [pltpu.SemaphoreType.DMA for _ in range(NBUF)],
    ],
    compiler_params=pltpu.CompilerParams(needs_layout_passes=False, use_tc_tiling_on_sc=False),
)
def _score_kernel(h_hbm, src_hbm, dst_hbm, out_hbm,
                  idx_s, idx_d, rows_s, rows_d, out_v, sems_s, sems_d):
    wid = lax.axis_index("s") * NC + lax.axis_index("c")
    wbase = pl.multiple_of(wid * E_PER_W, 8)
    iota = lax.iota(jnp.int32, L)

    pltpu.sync_copy(src_hbm.at[pl.ds(wbase, E_PER_W)], idx_s)
    pltpu.sync_copy(dst_hbm.at[pl.ds(wbase, E_PER_W)], idx_d)

    def fire(c, buf):
        off = pl.multiple_of(c * CH, 8)
        pltpu.async_copy(h_hbm.at[idx_s.at[pl.ds(off, CH)]], rows_s[buf],
                         sems_s[buf])
        pltpu.async_copy(h_hbm.at[idx_d.at[pl.ds(off, CH)]], rows_d[buf],
                         sems_d[buf])

    def drain(buf):
        pltpu.make_async_copy(h_hbm.at[idx_s.at[pl.ds(0, CH)]], rows_s[buf],
                              sems_s[buf]).wait()
        pltpu.make_async_copy(h_hbm.at[idx_d.at[pl.ds(0, CH)]], rows_d[buf],
                              sems_d[buf]).wait()

    def compute(c, buf):
        rs = rows_s[buf]
        rd = rows_d[buf]

        @pl.loop(0, G_PER_CH)
        def _group(g):
            edge = g * L + iota

            init = (iota, jnp.zeros((L,), jnp.float32),
                    jnp.zeros((L,), jnp.float32),
                    jnp.zeros((L,), jnp.float32),
                    jnp.zeros((L,), jnp.float32))

            @pl.loop(0, DW, init_carry=init, unroll=8)
            def _col(dcol, carry):
                colv, a0, a1, a2, a3 = carry
                aw = plsc.load_gather(rs, [edge, colv])
                bw = plsc.load_gather(rd, [edge, colv])
                s0, s1 = plsc.unpack(plsc.bitcast(aw, jnp.bfloat16),
                                     format=plsc.PackFormat.INTERLEAVED)
                d0, d1 = plsc.unpack(plsc.bitcast(bw, jnp.bfloat16),
                                     format=plsc.PackFormat.INTERLEAVED)
                return ((colv + 1) & (DW - 1), a2, a3,
                        a0 + s0 * d0, a1 + s1 * d1)

            _, a0, a1, a2, a3 = _col
            acc = (a0 + a1) + (a2 + a3)
            off = pl.multiple_of(c * CH + g * L, 8)
            out_v[pl.ds(off, L)] = acc

    for b in range(NBUF - 1):
        fire(b, b)

    @pl.loop(0, N_CH - 1, step=NBUF)
    def _chunk(c):
        for k in range(NBUF):
            nxt = c + k + NBUF - 1

            @pl.when(nxt < N_CH)
            def _():
                fire(nxt, (k + NBUF - 1) % NBUF)

            drain(k)
            compute(c + k, k)

    drain((N_CH - 1) % NBUF)
    compute(N_CH - 1, (N_CH - 1) % NBUF)

    pltpu.sync_copy(out_v, out_hbm.at[pl.ds(wbase, E_PER_W)])


def kernel(u_f, v_f, edge_index):
    h = jnp.concatenate([u_f, v_f], axis=0)
    hw = jax.lax.bitcast_convert_type(
        h.astype(jnp.bfloat16).reshape(N_NODES, DW, 2), jnp.int32)
    ei = edge_index.astype(jnp.int32)
    score = _score_kernel(hw, ei[0], ei[1])
    return score.reshape(E, 1)


# speedup vs baseline: 1.0978x; 1.0978x over previous
import functools

import jax
import jax.numpy as jnp
from jax import lax
from jax.experimental import pallas as pl
from jax.experimental.pallas import tpu as pltpu
from jax.experimental.pallas import tpu_sc as plsc

N_NODES = 10000
D = 128
E = 320000
NC = 2
NS = 16
L = 16
NW = NC * NS
E_PER_W = E // NW
CH = 80
DW = D // 2
N_CH = E_PER_W // CH
G_PER_CH = CH // L
NBUF = 4

_mesh = plsc.VectorSubcoreMesh(core_axis_name="c", subcore_axis_name="s")


@functools.partial(
    pl.kernel,
    out_type=jax.ShapeDtypeStruct((E,), jnp.float32),
    mesh=_mesh,
    scratch_types=[
        pltpu.VMEM((E_PER_W,), jnp.int32),
        pltpu.VMEM((E_PER_W,), jnp.int32),
        [pltpu.VMEM((CH, DW), jnp.int32) for _ in range(NBUF)],
        [pltpu.VMEM((CH, DW), jnp.int32) for _ in range(NBUF)],
        pltpu.VMEM((E_PER_W,), jnp.float32),
        [pltpu.SemaphoreType.DMA for _ in range(NBUF)],
        [pltpu.SemaphoreType.DMA for _ in range(NBUF)],
    ],
    compiler_params=pltpu.CompilerParams(needs_layout_passes=False, use_tc_tiling_on_sc=False),
)
def _score_kernel(h_hbm, src_hbm, dst_hbm, out_hbm,
                  idx_s, idx_d, rows_s, rows_d, out_v, sems_s, sems_d):
    wid = lax.axis_index("s") * NC + lax.axis_index("c")
    wbase = pl.multiple_of(wid * E_PER_W, 8)
    iota = lax.iota(jnp.int32, L)

    pltpu.sync_copy(src_hbm.at[pl.ds(wbase, E_PER_W)], idx_s)
    pltpu.sync_copy(dst_hbm.at[pl.ds(wbase, E_PER_W)], idx_d)

    def fire(c, buf):
        off = pl.multiple_of(c * CH, 8)
        pltpu.async_copy(h_hbm.at[idx_s.at[pl.ds(off, CH)]], rows_s[buf],
                         sems_s[buf])
        pltpu.async_copy(h_hbm.at[idx_d.at[pl.ds(off, CH)]], rows_d[buf],
                         sems_d[buf])

    def drain(buf):
        pltpu.make_async_copy(h_hbm.at[idx_s.at[pl.ds(0, CH)]], rows_s[buf],
                              sems_s[buf]).wait()
        pltpu.make_async_copy(h_hbm.at[idx_d.at[pl.ds(0, CH)]], rows_d[buf],
                              sems_d[buf]).wait()

    def compute(c, buf):
        rs = rows_s[buf]
        rd = rows_d[buf]

        @pl.loop(0, G_PER_CH)
        def _group(g):
            edge = g * L + iota

            init = tuple([iota] + [jnp.zeros((L,), jnp.float32)
                                   for _ in range(8)])

            @pl.loop(0, DW // 4, init_carry=init, unroll=2)
            def _col(dcol, carry):
                colv = carry[0]
                accs = list(carry[1:])
                for j in range(4):
                    aw = plsc.load_gather(rs, [edge, colv])
                    bw = plsc.load_gather(rd, [edge, colv])
                    p = (plsc.bitcast(aw, jnp.bfloat16)
                         * plsc.bitcast(bw, jnp.bfloat16))
                    p0, p1 = plsc.unpack(p, format=plsc.PackFormat.INTERLEAVED)
                    accs[2 * j] = accs[2 * j] + p0
                    accs[2 * j + 1] = accs[2 * j + 1] + p1
                    colv = colv + (16 if j < 3 else -47)
                return tuple([colv] + accs)

            accs = _col[1:]
            acc = (((accs[0] + accs[1]) + (accs[2] + accs[3]))
                   + ((accs[4] + accs[5]) + (accs[6] + accs[7])))
            off = pl.multiple_of(c * CH + g * L, 8)
            out_v[pl.ds(off, L)] = acc

    for b in range(NBUF - 1):
        fire(b, b)

    @pl.loop(0, N_CH - 1, step=NBUF)
    def _chunk(c):
        for k in range(NBUF):
            nxt = c + k + NBUF - 1

            @pl.when(nxt < N_CH)
            def _():
                fire(nxt, (k + NBUF - 1) % NBUF)

            drain(k)
            compute(c + k, k)

    drain((N_CH - 1) % NBUF)
    compute(N_CH - 1, (N_CH - 1) % NBUF)

    pltpu.sync_copy(out_v, out_hbm.at[pl.ds(wbase, E_PER_W)])


def kernel(u_f, v_f, edge_index):
    h = jnp.concatenate([u_f, v_f], axis=0)
    hw = jax.lax.bitcast_convert_type(
        h.astype(jnp.bfloat16).reshape(N_NODES, DW, 2), jnp.int32)
    ei = edge_index.astype(jnp.int32)
    score = _score_kernel(hw, ei[0], ei[1])
    return score.reshape(E, 1)


# speedup vs baseline: 1.1284x; 1.0278x over previous
import functools

import jax
import jax.numpy as jnp
from jax import lax
from jax.experimental import pallas as pl
from jax.experimental.pallas import tpu as pltpu
from jax.experimental.pallas import tpu_sc as plsc

N_NODES = 10000
D = 128
E = 320000
NC = 2
NS = 16
L = 16
NW = NC * NS
E_PER_W = E // NW
CH = 80
DW = D // 2
N_CH = E_PER_W // CH
G_PER_CH = CH // L
NBUF = 4

_mesh = plsc.VectorSubcoreMesh(core_axis_name="c", subcore_axis_name="s")


@functools.partial(
    pl.kernel,
    out_type=jax.ShapeDtypeStruct((E,), jnp.float32),
    mesh=_mesh,
    scratch_types=[
        pltpu.VMEM((E_PER_W,), jnp.int32),
        pltpu.VMEM((E_PER_W,), jnp.int32),
        [pltpu.VMEM((CH, DW), jnp.int32) for _ in range(NBUF)],
        [pltpu.VMEM((CH, DW), jnp.int32) for _ in range(NBUF)],
        pltpu.VMEM((E_PER_W,), jnp.float32),
        [pltpu.SemaphoreType.DMA for _ in range(NBUF)],
        [pltpu.SemaphoreType.DMA for _ in range(NBUF)],
    ],
    compiler_params=pltpu.CompilerParams(needs_layout_passes=False, use_tc_tiling_on_sc=False),
)
def _score_kernel(h_hbm, src_hbm, dst_hbm, out_hbm,
                  idx_s, idx_d, rows_s, rows_d, out_v, sems_s, sems_d):
    wid = lax.axis_index("s") * NC + lax.axis_index("c")
    wbase = pl.multiple_of(wid * E_PER_W, 8)
    iota = lax.iota(jnp.int32, L)

    pltpu.sync_copy(src_hbm.at[pl.ds(wbase, E_PER_W)], idx_s)
    pltpu.sync_copy(dst_hbm.at[pl.ds(wbase, E_PER_W)], idx_d)

    def fire(c, buf):
        off = pl.multiple_of(c * CH, 8)
        pltpu.async_copy(h_hbm.at[idx_s.at[pl.ds(off, CH)]], rows_s[buf],
                         sems_s[buf])
        pltpu.async_copy(h_hbm.at[idx_d.at[pl.ds(off, CH)]], rows_d[buf],
                         sems_d[buf])

    def drain(buf):
        pltpu.make_async_copy(h_hbm.at[idx_s.at[pl.ds(0, CH)]], rows_s[buf],
                              sems_s[buf]).wait()
        pltpu.make_async_copy(h_hbm.at[idx_d.at[pl.ds(0, CH)]], rows_d[buf],
                              sems_d[buf]).wait()

    def compute(c, buf):
        rs = rows_s[buf]
        rd = rows_d[buf]

        @pl.loop(0, G_PER_CH)
        def _group(g):
            edge = g * L + iota

            init = tuple([iota] + [jnp.zeros((L,), jnp.float32)
                                   for _ in range(8)])

            @pl.loop(0, DW, init_carry=init, unroll=8)
            def _col(dcol, carry):
                colv = carry[0]
                accs = carry[1:]
                aw = plsc.load_gather(rs, [edge, colv])
                bw = plsc.load_gather(rd, [edge, colv])
                p = (plsc.bitcast(aw, jnp.bfloat16)
                     * plsc.bitcast(bw, jnp.bfloat16))
                p0, p1 = plsc.unpack(p, format=plsc.PackFormat.INTERLEAVED)
                return tuple([(colv + 1) & (DW - 1)] + list(accs[2:])
                             + [accs[0] + p0, accs[1] + p1])

            accs = _col[1:]
            acc = (((accs[0] + accs[1]) + (accs[2] + accs[3]))
                   + ((accs[4] + accs[5]) + (accs[6] + accs[7])))
            off = pl.multiple_of(c * CH + g * L, 8)
            out_v[pl.ds(off, L)] = acc

    for b in range(NBUF - 1):
        fire(b, b)

    @pl.loop(0, N_CH - 1, step=NBUF)
    def _chunk(c):
        for k in range(NBUF):
            nxt = c + k + NBUF - 1

            @pl.when(nxt < N_CH)
            def _():
                fire(nxt, (k + NBUF - 1) % NBUF)

            drain(k)
            compute(c + k, k)

    drain((N_CH - 1) % NBUF)
    compute(N_CH - 1, (N_CH - 1) % NBUF)

    pltpu.sync_copy(out_v, out_hbm.at[pl.ds(wbase, E_PER_W)])


def kernel(u_f, v_f, edge_index):
    h = jnp.concatenate([u_f, v_f], axis=0)
    hw = jax.lax.bitcast_convert_type(
        h.astype(jnp.bfloat16).reshape(N_NODES, DW, 2), jnp.int32)
    ei = edge_index.astype(jnp.int32)
    score = _score_kernel(hw, ei[0], ei[1])
    return score.reshape(E, 1)
